# baseline (speedup 1.0000x reference)
import functools

import jax
import jax.numpy as jnp
from jax import lax
from jax.experimental import pallas as pl
from jax.experimental.pallas import tpu as pltpu
from jax.experimental.pallas import tpu_sc as plsc

N = 10000
E = 320000
H = 128

NC = 2
NS = 16
NW = NC * NS

EPW = E // NW
C = 80
NCHUNK = EPW // C
N_PAD = 10112
RPT = N_PAD // NS
NBUF = 4

def _sc_aggregate(meta_hbm, hidden_hbm, out_hbm,
                  pk0, pk1, pk2, pk3, src0, src1, src2, src3,
                  dst0, dst1, dst2, dst3,
                  rows0, rows1, rows2, rows3, acc,
                  sem_pk, sem_g, sem_s):
    c = lax.axis_index("c")
    s = lax.axis_index("s")
    wid = c * NS + s
    pk = (pk0, pk1, pk2, pk3)
    src = (src0, src1, src2, src3)
    dst = (dst0, dst1, dst2, dst3)
    rows = (rows0, rows1, rows2, rows3)

    mbase = wid * NCHUNK * 2 * C

    def issue_packed(b, k):
        pltpu.async_copy(meta_hbm.at[pl.ds(mbase + k * 2 * C, 2 * C)],
                         pk[b], sem_pk[b])

    def wait_packed(b):
        pltpu.make_async_copy(meta_hbm.at[pl.ds(0, 2 * C)], pk[b],
                              sem_pk[b]).wait()

    def unpack_idx(b):
        for i in range(C // 16):
            sl = pl.ds(i * 16, 16)
            v = pk[b][sl]
            dst[b][sl] = jnp.bitwise_and(v, 16383)
            src[b][sl] = jnp.right_shift(v, 14)

    def issue_gather(b):
        pltpu.async_copy(hidden_hbm.at[dst[b]], rows[b], sem_g[b])

    def wait_gather(b):
        pltpu.make_async_copy(hidden_hbm.at[dst[b]], rows[b], sem_g[b]).wait()

    def issue_scatter(b, f):
        pltpu.async_copy(rows[b], acc.at[src[b]], sem_s[b], add=True)

    def wait_scatter(b, f):
        pltpu.make_async_copy(rows[b], acc.at[src[b]], sem_s[b]).wait()

    def scale(b, f):
        def _blk(i, _):
            pv = lax.bitcast_convert_type(pk[b][pl.ds(C + i * 16, 16)], jnp.float32)
            for r in range(16):
                row = i * 16 + r
                p = pv[r]
                for g in range(H // 16):
                    sl = pl.ds(g * 16, 16)
                    rows[b][row, sl] = rows[b][row, sl] * p
            return 0
        lax.fori_loop(0, C // 16, _blk, 0)

    def _zero_blk(r, _):
        for g in range(H // 16):
            rows0[r, pl.ds(g * 16, 16)] = jnp.zeros((16,), jnp.float32)
        return 0
    lax.fori_loop(0, C, _zero_blk, 0)
    base_row = s * RPT
    for j in range(RPT // C):
        pltpu.sync_copy(rows0, acc.at[pl.ds(base_row + j * C, C)])
    pltpu.sync_copy(rows0.at[pl.ds(0, RPT % C)],
                    acc.at[pl.ds(base_row + (RPT // C) * C, RPT % C)])
    plsc.subcore_barrier()

    issue_packed(0, 0)
    issue_packed(1, 1)
    issue_packed(2, 2)
    for k in (0, 1):
        wait_packed(k)
        unpack_idx(k)
        issue_gather(k)
    for k in (0, 1):
        b2, b3 = (k + 2) % NBUF, (k + 3) % NBUF
        wait_packed(b2)
        unpack_idx(b2)
        issue_gather(b2)
        issue_packed(b3, k + 3)
        wait_gather(k)
        scale(k, k % 2)
        issue_scatter(k, k % 2)

    def _group(k4, _):
        for bb in range(NBUF):
            k = 2 + k4 * NBUF + bb
            b = (2 + bb) % NBUF
            b2 = (b + 2) % NBUF
            b3 = (b + 3) % NBUF
            f = bb % 2
            wait_scatter(b2, f)
            wait_packed(b2)
            unpack_idx(b2)
            issue_gather(b2)
            issue_packed(b3, k + 3)
            wait_gather(b)
            scale(b, f)
            issue_scatter(b, f)
        return 0
    lax.fori_loop(0, (NCHUNK - 5) // NBUF, _group, 0)

    wait_scatter(0, 0)
    wait_packed(0)
    unpack_idx(0)
    issue_gather(0)
    wait_gather(2)
    scale(2, 0)
    issue_scatter(2, 0)
    wait_scatter(1, 1)
    wait_gather(3)
    scale(3, 1)
    issue_scatter(3, 1)
    wait_scatter(2, 0)
    wait_gather(0)
    scale(0, 0)
    issue_scatter(0, 0)
    wait_scatter(3, 1)
    wait_scatter(0, 0)
    plsc.subcore_barrier()

    for j in range(RPT // C):
        r0 = base_row + j * C
        pltpu.sync_copy(acc.at[pl.ds(r0, C)], rows0)
        pltpu.sync_copy(rows0, out_hbm.at[c, pl.ds(r0, C)])
    r0 = base_row + (RPT // C) * C
    rem = RPT % C
    pltpu.sync_copy(acc.at[pl.ds(r0, rem)], rows0.at[pl.ds(0, rem)])
    pltpu.sync_copy(rows0.at[pl.ds(0, rem)], out_hbm.at[c, pl.ds(r0, rem)])


_sc_call = functools.partial(
    pl.kernel,
    out_type=jax.ShapeDtypeStruct((NC, N_PAD, H), jnp.float32),
    mesh=plsc.VectorSubcoreMesh(core_axis_name="c", subcore_axis_name="s"),
    scratch_types=(
        [pltpu.VMEM((2 * C,), jnp.int32) for _ in range(NBUF)]
        + [pltpu.VMEM((C,), jnp.int32) for _ in range(NBUF)]
        + [pltpu.VMEM((C,), jnp.int32) for _ in range(NBUF)]
        + [pltpu.VMEM((C, H), jnp.float32) for _ in range(NBUF)]
        + [pltpu.VMEM_SHARED((N_PAD, H), jnp.float32)]
        + [[pltpu.SemaphoreType.DMA] * NBUF,
           [pltpu.SemaphoreType.DMA] * NBUF,
           [pltpu.SemaphoreType.DMA] * NBUF]
    ),
)(_sc_aggregate)


def _tc_project(partials_ref, hs_ref, wrow_ref, wptr_ref, bias_ref, out_ref):
    agg = partials_ref[0, pl.ds(0, N)] + partials_ref[1, pl.ds(0, N)]
    enc_ptr = lax.dot_general(agg, wptr_ref[...], (((1,), (1,)), ((), ())),
                              preferred_element_type=jnp.float32)
    out_ref[...] = enc_ptr + hs_ref[...] * wrow_ref[...] + bias_ref[...]


def kernel(hint_scalar, hint_pointer, current_hidden, edge_index, step,
           W_scalar, b_scalar, W_ptr, b_ptr):
    src = edge_index[0]
    dst = edge_index[1]
    packed = src * 16384 + dst
    prob = jnp.take(hint_pointer, step, axis=1)
    prob_bits = jax.lax.bitcast_convert_type(prob, jnp.int32)
    meta = jnp.concatenate(
        [packed.reshape(-1, 1, C), prob_bits.reshape(-1, 1, C)],
        axis=1).reshape(-1)
    hs = jnp.take(hint_scalar, step, axis=1)[:, None]
    partials = _sc_call(meta, current_hidden)

    wrow = W_scalar.reshape(1, H)
    bias = (b_scalar + b_ptr).reshape(1, H)
    out = pl.pallas_call(
        _tc_project,
        out_shape=jax.ShapeDtypeStruct((N, H), jnp.float32),
    )(partials, hs, wrow, W_ptr, bias)
    return out


# speedup vs baseline: 1.1657x; 1.1657x over previous
import functools

import jax
import jax.numpy as jnp
from jax import lax
from jax.experimental import pallas as pl
from jax.experimental.pallas import tpu as pltpu
from jax.experimental.pallas import tpu_sc as plsc

N = 10000
E = 320000
H = 128

NC = 2
NS = 16
NW = NC * NS

EPW = E // NW
C = 80
NCHUNK = EPW // C
N_PAD = 10112
RPT = N_PAD // NS
NBUF = 4

_SUB = [C] * (RPT // C) + ([RPT % C] if RPT % C else [])


def _sc_aggregate(packed_hbm, prob_hbm, hidden_hbm, out_hbm,
                  pk0, pk1, pk2, pk3, src0, src1, src2, src3,
                  dst0, dst1, dst2, dst3, pr0, pr1, pr2, pr3,
                  rows0, rows1, rows2, rows3, acc,
                  sem_pk, sem_pr, sem_g, sem_s):
    c = lax.axis_index("c")
    s = lax.axis_index("s")
    wid = c * NS + s
    pk = (pk0, pk1, pk2, pk3)
    src = (src0, src1, src2, src3)
    dst = (dst0, dst1, dst2, dst3)
    pr = (pr0, pr1, pr2, pr3)
    rows = (rows0, rows1, rows2, rows3)

    ebase = wid * EPW

    def issue_packed(b, k):
        pltpu.async_copy(packed_hbm.at[pl.ds(ebase + k * C, C)], pk[b], sem_pk[b])

    def wait_packed(b):
        pltpu.make_async_copy(packed_hbm.at[pl.ds(0, C)], pk[b], sem_pk[b]).wait()

    def issue_prob(b, k):
        pltpu.async_copy(prob_hbm.at[pl.ds(ebase + k * C, C)], pr[b], sem_pr[b])

    def wait_prob(b):
        pltpu.make_async_copy(prob_hbm.at[pl.ds(0, C)], pr[b], sem_pr[b]).wait()

    def unpack_idx(b):
        for i in range(C // 16):
            sl = pl.ds(i * 16, 16)
            v = pk[b][sl]
            dst[b][sl] = jnp.bitwise_and(v, 16383)
            src[b][sl] = jnp.right_shift(v, 14)

    def issue_gather(b):
        pltpu.async_copy(hidden_hbm.at[dst[b]], rows[b], sem_g[b])

    def wait_gather(b):
        pltpu.make_async_copy(hidden_hbm.at[dst[b]], rows[b], sem_g[b]).wait()

    def issue_scatter(b):
        pltpu.async_copy(rows[b], acc.at[src[b]], sem_s[b], add=True)

    def wait_scatter(b):
        pltpu.make_async_copy(rows[b], acc.at[src[b]], sem_s[b]).wait()

    def scale(b):
        def _blk(i, _):
            pv = pr[b][pl.ds(i * 16, 16)]
            for r in range(16):
                row = i * 16 + r
                p = pv[r]
                for g in range(H // 16):
                    sl = pl.ds(g * 16, 16)
                    rows[b][row, sl] = rows[b][row, sl] * p
            return 0
        lax.fori_loop(0, C // 16, _blk, 0)

    def _zero_blk(r, _):
        for g in range(H // 16):
            rows0[r, pl.ds(g * 16, 16)] = jnp.zeros((16,), jnp.float32)
        return 0
    lax.fori_loop(0, C, _zero_blk, 0)
    base_row = s * RPT
    off = 0
    for sz in _SUB:
        pltpu.async_copy(rows0.at[pl.ds(0, sz)],
                         acc.at[pl.ds(base_row + off, sz)], sem_g[0])
        off += sz
    off = 0
    for sz in _SUB:
        pltpu.make_async_copy(rows0.at[pl.ds(0, sz)],
                              acc.at[pl.ds(base_row + off, sz)], sem_g[0]).wait()
        off += sz
    plsc.subcore_barrier()

    issue_packed(0, 0)
    issue_packed(1, 1)
    issue_packed(2, 2)
    for k in (0, 1):
        wait_packed(k)
        unpack_idx(k)
        issue_gather(k)
        issue_prob(k, k)
    for k in (0, 1):
        b2, b3 = (k + 2) % NBUF, (k + 3) % NBUF
        wait_packed(b2)
        unpack_idx(b2)
        issue_gather(b2)
        issue_prob(b2, k + 2)
        issue_packed(b3, k + 3)
        wait_gather(k)
        wait_prob(k)
        scale(k)
        issue_scatter(k)

    def _group(k4, _):
        for bb in range(NBUF):
            k = 2 + k4 * NBUF + bb
            b = (2 + bb) % NBUF
            b2 = (b + 2) % NBUF
            b3 = (b + 3) % NBUF
            wait_scatter(b2)
            wait_packed(b2)
            unpack_idx(b2)
            issue_gather(b2)
            issue_prob(b2, k + 2)
            issue_packed(b3, k + 3)
            wait_gather(b)
            wait_prob(b)
            scale(b)
            issue_scatter(b)
        return 0
    lax.fori_loop(0, (NCHUNK - 5) // NBUF, _group, 0)

    wait_scatter(0)
    wait_packed(0)
    unpack_idx(0)
    issue_gather(0)
    issue_prob(0, NCHUNK - 1)
    wait_gather(2)
    wait_prob(2)
    scale(2)
    issue_scatter(2)
    wait_scatter(1)
    wait_gather(3)
    wait_prob(3)
    scale(3)
    issue_scatter(3)
    wait_scatter(2)
    wait_gather(0)
    wait_prob(0)
    scale(0)
    issue_scatter(0)
    wait_scatter(3)
    wait_scatter(0)
    plsc.subcore_barrier()

    offs = []
    off = 0
    for sz in _SUB:
        offs.append(off)
        off += sz
    for j, sz in enumerate(_SUB):
        b = j % NBUF
        if j >= NBUF:
            psz = _SUB[j - NBUF]
            pltpu.make_async_copy(
                rows[b].at[pl.ds(0, psz)],
                out_hbm.at[c, pl.ds(base_row + offs[j - NBUF], psz)],
                sem_s[b]).wait()
        pltpu.sync_copy(acc.at[pl.ds(base_row + offs[j], sz)],
                        rows[b].at[pl.ds(0, sz)])
        pltpu.async_copy(rows[b].at[pl.ds(0, sz)],
                         out_hbm.at[c, pl.ds(base_row + offs[j], sz)],
                         sem_s[b])
    for j in range(len(_SUB) - NBUF, len(_SUB)):
        b = j % NBUF
        sz = _SUB[j]
        pltpu.make_async_copy(
            rows[b].at[pl.ds(0, sz)],
            out_hbm.at[c, pl.ds(base_row + offs[j], sz)],
            sem_s[b]).wait()


_sc_call = functools.partial(
    pl.kernel,
    out_type=jax.ShapeDtypeStruct((NC, N_PAD, H), jnp.float32),
    mesh=plsc.VectorSubcoreMesh(core_axis_name="c", subcore_axis_name="s"),
    scratch_types=(
        [pltpu.VMEM((C,), jnp.int32) for _ in range(NBUF)]
        + [pltpu.VMEM((C,), jnp.int32) for _ in range(NBUF)]
        + [pltpu.VMEM((C,), jnp.int32) for _ in range(NBUF)]
        + [pltpu.VMEM((C,), jnp.float32) for _ in range(NBUF)]
        + [pltpu.VMEM((C, H), jnp.float32) for _ in range(NBUF)]
        + [pltpu.VMEM_SHARED((N_PAD, H), jnp.float32)]
        + [[pltpu.SemaphoreType.DMA] * NBUF,
           [pltpu.SemaphoreType.DMA] * NBUF,
           [pltpu.SemaphoreType.DMA] * NBUF,
           [pltpu.SemaphoreType.DMA] * NBUF]
    ),
)(_sc_aggregate)


def _tc_project(partials_ref, hs_ref, wrow_ref, wptr_ref, bias_ref, out_ref):
    agg = partials_ref[0, pl.ds(0, N)] + partials_ref[1, pl.ds(0, N)]
    enc_ptr = lax.dot_general(agg, wptr_ref[...], (((1,), (1,)), ((), ())),
                              preferred_element_type=jnp.float32)
    out_ref[...] = enc_ptr + hs_ref[...] * wrow_ref[...] + bias_ref[...]


def kernel(hint_scalar, hint_pointer, current_hidden, edge_index, step,
           W_scalar, b_scalar, W_ptr, b_ptr):
    src = edge_index[0]
    dst = edge_index[1]
    packed = src * 16384 + dst
    prob = jnp.take(hint_pointer, step, axis=1)
    hs = jnp.take(hint_scalar, step, axis=1)[:, None]
    partials = _sc_call(packed, prob, current_hidden)

    wrow = W_scalar.reshape(1, H)
    bias = (b_scalar + b_ptr).reshape(1, H)
    out = pl.pallas_call(
        _tc_project,
        out_shape=jax.ShapeDtypeStruct((N, H), jnp.float32),
    )(partials, hs, wrow, W_ptr, bias)
    return out


# speedup vs baseline: 1.1675x; 1.0015x over previous
import functools

import jax
import jax.numpy as jnp
from jax import lax
from jax.experimental import pallas as pl
from jax.experimental.pallas import tpu as pltpu
from jax.experimental.pallas import tpu_sc as plsc

N = 10000
E = 320000
H = 128

NC = 2
NS = 16
NW = NC * NS

EPW = E // NW
C = 80
NCHUNK = EPW // C
N_PAD = 10112
RPT = N_PAD // NS
NBUF = 4

_SUB = [C] * (RPT // C) + ([RPT % C] if RPT % C else [])


def _sc_aggregate(packed_hbm, prob_hbm, hidden_hbm, out_hbm,
                  pk0, pk1, pk2, pk3, src0, src1, src2, src3,
                  dst0, dst1, dst2, dst3, pr0, pr1, pr2, pr3,
                  rows0, rows1, rows2, rows3, acc,
                  sem_pk, sem_pr, sem_g, sem_s):
    c = lax.axis_index("c")
    s = lax.axis_index("s")
    wid = c * NS + s
    pk = (pk0, pk1, pk2, pk3)
    src = (src0, src1, src2, src3)
    dst = (dst0, dst1, dst2, dst3)
    pr = (pr0, pr1, pr2, pr3)
    rows = (rows0, rows1, rows2, rows3)

    ebase = wid * EPW

    def issue_packed(b, k):
        pltpu.async_copy(packed_hbm.at[pl.ds(ebase + k * C, C)], pk[b], sem_pk[b])

    def wait_packed(b):
        pltpu.make_async_copy(packed_hbm.at[pl.ds(0, C)], pk[b], sem_pk[b]).wait()

    def issue_prob(b, k):
        pltpu.async_copy(prob_hbm.at[pl.ds(ebase + k * C, C)], pr[b], sem_pr[b])

    def wait_prob(b):
        pltpu.make_async_copy(prob_hbm.at[pl.ds(0, C)], pr[b], sem_pr[b]).wait()

    def unpack_idx(b):
        for i in range(C // 16):
            sl = pl.ds(i * 16, 16)
            v = pk[b][sl]
            dst[b][sl] = jnp.bitwise_and(v, 16383)
            src[b][sl] = jnp.right_shift(v, 14)

    def issue_gather(b):
        pltpu.async_copy(hidden_hbm.at[dst[b]], rows[b], sem_g[b])

    def wait_gather(b):
        pltpu.make_async_copy(hidden_hbm.at[dst[b]], rows[b], sem_g[b]).wait()

    def issue_scatter(b):
        pltpu.async_copy(rows[b], acc.at[src[b]], sem_s[b], add=True)

    def wait_scatter(b):
        pltpu.make_async_copy(rows[b], acc.at[src[b]], sem_s[b]).wait()

    def scale(b):
        def _blk(i, _):
            pv = pr[b][pl.ds(i * 16, 16)]
            for r in range(16):
                row = i * 16 + r
                p = pv[r]
                for g in range(H // 16):
                    sl = pl.ds(g * 16, 16)
                    rows[b][row, sl] = rows[b][row, sl] * p
            return 0
        lax.fori_loop(0, C // 16, _blk, 0)

    def _zero_blk(r, _):
        for g in range(H // 16):
            rows0[r, pl.ds(g * 16, 16)] = jnp.zeros((16,), jnp.float32)
        return 0
    lax.fori_loop(0, C, _zero_blk, 0)
    base_row = s * RPT
    off = 0
    for sz in _SUB:
        pltpu.async_copy(rows0.at[pl.ds(0, sz)],
                         acc.at[pl.ds(base_row + off, sz)], sem_g[0])
        off += sz
    off = 0
    for sz in _SUB:
        pltpu.make_async_copy(rows0.at[pl.ds(0, sz)],
                              acc.at[pl.ds(base_row + off, sz)], sem_g[0]).wait()
        off += sz
    plsc.subcore_barrier()

    issue_packed(0, 0)
    issue_packed(1, 1)
    issue_packed(2, 2)
    for k in (0, 1):
        wait_packed(k)
        unpack_idx(k)
        issue_gather(k)
        issue_prob(k, k)
    for k in (0, 1):
        b2, b3 = (k + 2) % NBUF, (k + 3) % NBUF
        wait_packed(b2)
        unpack_idx(b2)
        issue_gather(b2)
        issue_prob(b2, k + 2)
        issue_packed(b3, k + 3)
        wait_gather(k)
        wait_prob(k)
        scale(k)
        issue_scatter(k)

    def _group(k4, _):
        for bb in range(NBUF):
            k = 2 + k4 * NBUF + bb
            b = (2 + bb) % NBUF
            b2 = (b + 2) % NBUF
            b3 = (b + 3) % NBUF
            wait_scatter(b2)
            wait_packed(b2)
            unpack_idx(b2)
            issue_gather(b2)
            issue_prob(b2, k + 2)
            issue_packed(b3, k + 3)
            wait_gather(b)
            wait_prob(b)
            scale(b)
            issue_scatter(b)
        return 0
    lax.fori_loop(0, (NCHUNK - 5) // NBUF, _group, 0)

    wait_scatter(0)
    wait_packed(0)
    unpack_idx(0)
    issue_gather(0)
    issue_prob(0, NCHUNK - 1)
    wait_gather(2)
    wait_prob(2)
    scale(2)
    issue_scatter(2)
    wait_scatter(1)
    wait_gather(3)
    wait_prob(3)
    scale(3)
    issue_scatter(3)
    wait_scatter(2)
    wait_gather(0)
    wait_prob(0)
    scale(0)
    issue_scatter(0)
    wait_scatter(3)
    wait_scatter(0)
    plsc.subcore_barrier()

    offs = []
    off = 0
    for sz in _SUB:
        offs.append(off)
        off += sz
    for j, sz in enumerate(_SUB):
        b = j % NBUF
        if j >= NBUF:
            psz = _SUB[j - NBUF]
            pltpu.make_async_copy(
                rows[b].at[pl.ds(0, psz)],
                out_hbm.at[c, pl.ds(base_row + offs[j - NBUF], psz)],
                sem_s[b]).wait()
        pltpu.sync_copy(acc.at[pl.ds(base_row + offs[j], sz)],
                        rows[b].at[pl.ds(0, sz)])
        pltpu.async_copy(rows[b].at[pl.ds(0, sz)],
                         out_hbm.at[c, pl.ds(base_row + offs[j], sz)],
                         sem_s[b])
    for j in range(len(_SUB) - NBUF, len(_SUB)):
        b = j % NBUF
        sz = _SUB[j]
        pltpu.make_async_copy(
            rows[b].at[pl.ds(0, sz)],
            out_hbm.at[c, pl.ds(base_row + offs[j], sz)],
            sem_s[b]).wait()


_sc_call = functools.partial(
    pl.kernel,
    out_type=jax.ShapeDtypeStruct((NC, N_PAD, H), jnp.float32),
    mesh=plsc.VectorSubcoreMesh(core_axis_name="c", subcore_axis_name="s"),
    compiler_params=pltpu.CompilerParams(allow_input_fusion=[0, 1]),
    scratch_types=(
        [pltpu.VMEM((C,), jnp.int32) for _ in range(NBUF)]
        + [pltpu.VMEM((C,), jnp.int32) for _ in range(NBUF)]
        + [pltpu.VMEM((C,), jnp.int32) for _ in range(NBUF)]
        + [pltpu.VMEM((C,), jnp.float32) for _ in range(NBUF)]
        + [pltpu.VMEM((C, H), jnp.float32) for _ in range(NBUF)]
        + [pltpu.VMEM_SHARED((N_PAD, H), jnp.float32)]
        + [[pltpu.SemaphoreType.DMA] * NBUF,
           [pltpu.SemaphoreType.DMA] * NBUF,
           [pltpu.SemaphoreType.DMA] * NBUF,
           [pltpu.SemaphoreType.DMA] * NBUF]
    ),
)(_sc_aggregate)


def _tc_project(partials_ref, hs_ref, wrow_ref, wptr_ref, bias_ref, out_ref):
    agg = partials_ref[0, pl.ds(0, N)] + partials_ref[1, pl.ds(0, N)]
    enc_ptr = lax.dot_general(agg, wptr_ref[...], (((1,), (1,)), ((), ())),
                              preferred_element_type=jnp.float32)
    out_ref[...] = enc_ptr + hs_ref[...] * wrow_ref[...] + bias_ref[...]


def kernel(hint_scalar, hint_pointer, current_hidden, edge_index, step,
           W_scalar, b_scalar, W_ptr, b_ptr):
    src = edge_index[0]
    dst = edge_index[1]
    packed = src * 16384 + dst
    prob = jnp.take(hint_pointer, step, axis=1)
    hs = jnp.take(hint_scalar, step, axis=1)[:, None]
    partials = _sc_call(packed, prob, current_hidden)

    wrow = W_scalar.reshape(1, H)
    bias = (b_scalar + b_ptr).reshape(1, H)
    out = pl.pallas_call(
        _tc_project,
        out_shape=jax.ShapeDtypeStruct((N, H), jnp.float32),
    )(partials, hs, wrow, W_ptr, bias)
    return out
